# Initial kernel scaffold; baseline (speedup 1.0000x reference)
import jax
import jax.numpy as jnp
from jax.experimental import pallas as pl


def kernel(A, D, observation):
    raise NotImplementedError("write your pallas kernel here")


# speedup vs baseline: 3.0539x; 3.0539x over previous
import jax
import jax.numpy as jnp
from jax.experimental import pallas as pl

N_STATES = 100000
N_SAMPLES = 1000
N_WARMUP = 100
STEPS_PER_SAMPLE = 5

LANES = 128
SUBROWS = 782
PADDED = SUBROWS * LANES

KS0 = 64467757
KS1 = 2916123636
KS2 = (KS0 ^ KS1 ^ 0x1BD11BDA) & 0xFFFFFFFF

_ROT_A = (13, 15, 26, 6)
_ROT_B = (17, 29, 16, 24)


def _sampler_kernel(lw_ref, counts_ref):
    p = pl.program_id(0)

    @pl.when(p == 0)
    def _init():
        counts_ref[...] = jnp.zeros_like(counts_ref)

    t = N_WARMUP + STEPS_PER_SAMPLE * p
    base = t * N_STATES
    i = jax.lax.broadcasted_iota(jnp.int32, (SUBROWS, LANES), 0)
    c = jax.lax.broadcasted_iota(jnp.int32, (SUBROWS, LANES), 1)
    flat = i * LANES + c
    ctr = (flat + base).astype(jnp.uint32)

    ks = (KS0, KS1, KS2)
    x0 = jnp.full((SUBROWS, LANES), jnp.uint32(KS0), dtype=jnp.uint32)
    x1 = ctr + jnp.uint32(KS1)
    rots = (_ROT_A, _ROT_B)
    for rnd in range(5):
        for r in rots[rnd % 2]:
            x0 = x0 + x1
            x1 = jax.lax.shift_left(x1, jnp.uint32(r)) | \
                jax.lax.shift_right_logical(x1, jnp.uint32(32 - r))
            x1 = x0 ^ x1
        x0 = x0 + jnp.uint32(ks[(rnd + 1) % 3])
        x1 = x1 + jnp.uint32((ks[(rnd + 2) % 3] + rnd + 1) & 0xFFFFFFFF)
    bits = x0 ^ x1

    fb = jax.lax.shift_right_logical(bits, jnp.uint32(9)) | jnp.uint32(0x3F800000)
    u = jax.lax.bitcast_convert_type(fb, jnp.float32) - jnp.float32(1.0)
    tiny = jnp.float32(jnp.finfo(jnp.float32).tiny)
    u = jnp.maximum(tiny, u + tiny)
    g = -jnp.log(-jnp.log(u))

    score = g + lw_ref[...]
    m = jnp.max(score)
    winner = jnp.min(jnp.where(score == m, flat, jnp.int32(2**30)))
    counts_ref[...] += (flat == winner).astype(jnp.float32)


def _draw_counts(lw_pad):
    return pl.pallas_call(
        _sampler_kernel,
        grid=(N_SAMPLES,),
        in_specs=[pl.BlockSpec((SUBROWS, LANES), lambda p: (0, 0))],
        out_specs=pl.BlockSpec((SUBROWS, LANES), lambda p: (0, 0)),
        out_shape=jax.ShapeDtypeStruct((SUBROWS, LANES), jnp.float32),
    )(lw_pad)


def kernel(A, D, observation):
    likelihood = A[observation, :]
    posterior_weights = likelihood * D
    posterior_weights = posterior_weights / (jnp.sum(posterior_weights) + 1e-16)
    log_weights = jnp.log(posterior_weights + 1e-16)
    lw_pad = jnp.concatenate(
        [log_weights,
         jnp.full((PADDED - N_STATES,), -jnp.inf, dtype=jnp.float32)]
    ).reshape(SUBROWS, LANES)

    counts = _draw_counts(lw_pad)

    counts_flat = counts.reshape(-1)[:N_STATES]
    posterior_estimate = counts_flat / float(N_SAMPLES)
    return posterior_estimate / (jnp.sum(posterior_estimate) + 1e-16)


# speedup vs baseline: 4.1106x; 1.3460x over previous
import jax
import jax.numpy as jnp
from jax.experimental import pallas as pl

N_STATES = 100000
N_SAMPLES = 1000
N_WARMUP = 100
STEPS_PER_SAMPLE = 5

LANES = 128
CHUNK_SUB = 80
N_CHUNKS = 10
SUBROWS = CHUNK_SUB * N_CHUNKS
PADDED = SUBROWS * LANES

KS0 = 64467757
KS1 = 2916123636
KS2 = (KS0 ^ KS1 ^ 0x1BD11BDA) & 0xFFFFFFFF

_ROT_A = (13, 15, 26, 6)
_ROT_B = (17, 29, 16, 24)


def _sampler_kernel(lw_ref, counts_ref):
    p = pl.program_id(0)

    @pl.when(p == 0)
    def _init():
        counts_ref[...] = jnp.zeros_like(counts_ref)

    t = N_WARMUP + STEPS_PER_SAMPLE * p
    base = t * N_STATES

    i = jax.lax.broadcasted_iota(jnp.int32, (CHUNK_SUB, LANES), 0)
    c = jax.lax.broadcasted_iota(jnp.int32, (CHUNK_SUB, LANES), 1)
    flat0 = i * LANES + c

    def chunk(k, carry):
        best_v, best_j = carry
        flat = flat0 + k * (CHUNK_SUB * LANES)
        ctr = (flat + base).astype(jnp.uint32)

        ks = (KS0, KS1, KS2)
        x0 = jnp.full((CHUNK_SUB, LANES), jnp.uint32(KS0), dtype=jnp.uint32)
        x1 = ctr + jnp.uint32(KS1)
        rots = (_ROT_A, _ROT_B)
        for rnd in range(5):
            for r in rots[rnd % 2]:
                x0 = x0 + x1
                x1 = jax.lax.shift_left(x1, jnp.uint32(r)) | \
                    jax.lax.shift_right_logical(x1, jnp.uint32(32 - r))
                x1 = x0 ^ x1
            x0 = x0 + jnp.uint32(ks[(rnd + 1) % 3])
            x1 = x1 + jnp.uint32((ks[(rnd + 2) % 3] + rnd + 1) & 0xFFFFFFFF)
        bits = x0 ^ x1

        fb = jax.lax.shift_right_logical(bits, jnp.uint32(9)) | \
            jnp.uint32(0x3F800000)
        u = jax.lax.bitcast_convert_type(fb, jnp.float32) - jnp.float32(1.0)
        tiny = jnp.float32(jnp.finfo(jnp.float32).tiny)
        u = jnp.maximum(tiny, u + tiny)
        g = -jnp.log(-jnp.log(u))

        score = g + lw_ref[pl.ds(k * CHUNK_SUB, CHUNK_SUB), :]
        upd = score > best_v
        return (jnp.where(upd, score, best_v), jnp.where(upd, flat, best_j))

    neg_inf = jnp.full((CHUNK_SUB, LANES), -jnp.inf, dtype=jnp.float32)
    zero_j = jnp.zeros((CHUNK_SUB, LANES), dtype=jnp.int32)
    best_v, best_j = jax.lax.fori_loop(0, N_CHUNKS, chunk, (neg_inf, zero_j))

    m = jnp.max(best_v)
    winner = jnp.min(jnp.where(best_v == m, best_j, jnp.int32(2**30)))

    i_all = jax.lax.broadcasted_iota(jnp.int32, (SUBROWS, LANES), 0)
    c_all = jax.lax.broadcasted_iota(jnp.int32, (SUBROWS, LANES), 1)
    flat_all = i_all * LANES + c_all
    counts_ref[...] += (flat_all == winner).astype(jnp.float32)


def _draw_counts(lw_pad):
    return pl.pallas_call(
        _sampler_kernel,
        grid=(N_SAMPLES,),
        in_specs=[pl.BlockSpec((SUBROWS, LANES), lambda p: (0, 0))],
        out_specs=pl.BlockSpec((SUBROWS, LANES), lambda p: (0, 0)),
        out_shape=jax.ShapeDtypeStruct((SUBROWS, LANES), jnp.float32),
    )(lw_pad)


def kernel(A, D, observation):
    likelihood = A[observation, :]
    posterior_weights = likelihood * D
    posterior_weights = posterior_weights / (jnp.sum(posterior_weights) + 1e-16)
    log_weights = jnp.log(posterior_weights + 1e-16)
    lw_pad = jnp.concatenate(
        [log_weights,
         jnp.full((PADDED - N_STATES,), -jnp.inf, dtype=jnp.float32)]
    ).reshape(SUBROWS, LANES)

    counts = _draw_counts(lw_pad)

    counts_flat = counts.reshape(-1)[:N_STATES]
    posterior_estimate = counts_flat / float(N_SAMPLES)
    return posterior_estimate / (jnp.sum(posterior_estimate) + 1e-16)
